# Initial kernel scaffold; baseline (speedup 1.0000x reference)
import functools
import math

import jax
import jax.numpy as jnp
from jax import lax
from jax.experimental import pallas as pl
from jax.experimental.pallas import tpu as pltpu
from jax.experimental.pallas import tpu_sc as plsc

V = 100000
E = 128
B = 128
L = 32
NR = 100
N = B * L
NW = 32
TPW = N // NW
NSP = 128

LOG_V = math.log(V)
LOG_NR = math.log(NR)
LOG_S = math.log(V * (V + 1) / 2.0)

_mesh = plsc.VectorSubcoreMesh(core_axis_name="c", subcore_axis_name="s")


@functools.partial(
    pl.kernel,
    mesh=_mesh,
    out_type=[
        jax.ShapeDtypeStruct((N, E), jnp.float32),
        jax.ShapeDtypeStruct((N, 1), jnp.float32),
        jax.ShapeDtypeStruct((NSP, E), jnp.float32),
        jax.ShapeDtypeStruct((NSP, 1), jnp.float32),
    ],
    scratch_types=[
        pltpu.VMEM((TPW,), jnp.int32),
        pltpu.VMEM((TPW, E), jnp.float32),
        pltpu.VMEM((TPW, 1), jnp.float32),
        pltpu.VMEM((NSP,), jnp.int32),
        pltpu.VMEM((NSP, E), jnp.float32),
        pltpu.VMEM((NSP, 1), jnp.float32),
        pltpu.SemaphoreType.DMA,
        pltpu.SemaphoreType.DMA,
    ],
)
def _sc_gather(tgt_hbm, ns_hbm, emb_hbm, bias_hbm,
               tb_out, tbias_out, nb_out, nbias_out,
               idx_v, rows_v, tbias_v, nidx_v, nrows_v, nbias_v, sem, nsem):
    wid = lax.axis_index("s") * 2 + lax.axis_index("c")
    base = wid * TPW
    pltpu.sync_copy(tgt_hbm.at[pl.ds(base, TPW)], idx_v)
    cp_rows = pltpu.async_copy(emb_hbm.at[idx_v], rows_v, sem)
    cp_bias = pltpu.async_copy(bias_hbm.at[idx_v], tbias_v, sem)

    @pl.when(wid == 0)
    def _():
        pltpu.sync_copy(ns_hbm, nidx_v)
        pltpu.async_copy(emb_hbm.at[nidx_v], nrows_v, nsem).wait()
        pltpu.sync_copy(nrows_v, nb_out)

    @pl.when(wid == 1)
    def _():
        pltpu.sync_copy(ns_hbm, nidx_v)
        pltpu.async_copy(bias_hbm.at[nidx_v], nbias_v, nsem).wait()
        pltpu.sync_copy(nbias_v, nbias_out)

    cp_rows.wait()
    cp_bias.wait()
    pltpu.sync_copy(rows_v, tb_out.at[pl.ds(base, TPW)])
    pltpu.sync_copy(tbias_v, tbias_out.at[pl.ds(base, TPW)])


BLK = 512


def _tc_loss_body(inp_ref, tb_ref, tbias_ref, tgt_ref, nb_ref, nbias_ref,
                  ns_ref, out_ref):
    step = pl.program_id(0)

    @pl.when(step == 0)
    def _():
        out_ref[0, 0] = 0.0

    inp_b = inp_ref[...]
    tdot = jnp.sum(inp_b * tb_ref[...], axis=1, keepdims=True) + tbias_ref[...]
    tgt_f = tgt_ref[...].astype(jnp.float32)
    x_t = tdot - LOG_V - (jnp.log(tgt_f + 1.0) - LOG_S) - LOG_NR
    t_sum = jnp.sum(jnp.maximum(-x_t, 0.0)
                    + jnp.log(1.0 + jnp.exp(-jnp.abs(x_t))))

    ns_f = ns_ref[...].astype(jnp.float32)
    row_off = nbias_ref[...] - LOG_V - (jnp.log(ns_f + 1.0) - LOG_S) - LOG_NR
    scores = lax.dot_general(nb_ref[...], inp_b, (((1,), (1,)), ((), ())),
                             preferred_element_type=jnp.float32)
    x_n = scores + row_off
    sp = jnp.maximum(x_n, 0.0) + jnp.log(1.0 + jnp.exp(-jnp.abs(x_n)))
    mask = lax.broadcasted_iota(jnp.int32, (NSP, BLK), 0) < NR
    n_sum = jnp.sum(jnp.where(mask, sp, 0.0))

    out_ref[0, 0] += (t_sum + n_sum) * (1.0 / N)


_tc_loss = pl.pallas_call(
    _tc_loss_body,
    grid=(N // BLK,),
    in_specs=[
        pl.BlockSpec((BLK, E), lambda i: (i, 0)),
        pl.BlockSpec((BLK, E), lambda i: (i, 0)),
        pl.BlockSpec((BLK, 1), lambda i: (i, 0)),
        pl.BlockSpec((BLK, 1), lambda i: (i, 0)),
        pl.BlockSpec((NSP, E), lambda i: (0, 0)),
        pl.BlockSpec((NSP, 1), lambda i: (0, 0)),
        pl.BlockSpec((NSP, 1), lambda i: (0, 0)),
    ],
    out_specs=pl.BlockSpec((1, 1), lambda i: (0, 0)),
    out_shape=jax.ShapeDtypeStruct((1, 1), jnp.float32),
)


def kernel(target, inp, noise_samples, emb_weight, bias_weight):
    tgt = target.reshape(-1).astype(jnp.int32)
    ns = jnp.concatenate([noise_samples.astype(jnp.int32),
                          jnp.zeros((NSP - NR,), jnp.int32)])
    inp_flat = inp.reshape(N, E)
    tb, tbias, nb, nbias = _sc_gather(tgt, ns, emb_weight, bias_weight)
    out = _tc_loss(inp_flat, tb, tbias, tgt.reshape(N, 1), nb, nbias,
                   ns.reshape(NSP, 1))
    return out[0, 0]


# speedup vs baseline: 1.7273x; 1.7273x over previous
import functools
import math

import jax
import jax.numpy as jnp
from jax import lax
from jax.experimental import pallas as pl
from jax.experimental.pallas import tpu as pltpu
from jax.experimental.pallas import tpu_sc as plsc

V = 100000
E = 128
B = 128
L = 32
NR = 100
N = B * L
NW = 32
TPW = N // NW
NSP = 128
BROWS = (V + E - 1) // E

LOG_V = math.log(V)
LOG_NR = math.log(NR)
LOG_S = math.log(V * (V + 1) / 2.0)

_mesh = plsc.VectorSubcoreMesh(core_axis_name="c", subcore_axis_name="s")


def _bias_row_idx(idx_ref, out_ref, n):
    for g in range(n // 16):
        out_ref[pl.ds(16 * g, 16)] = lax.shift_right_logical(
            idx_ref[pl.ds(16 * g, 16)], 7)


@functools.partial(
    pl.kernel,
    mesh=_mesh,
    out_type=[
        jax.ShapeDtypeStruct((N, E), jnp.float32),
        jax.ShapeDtypeStruct((N, E), jnp.float32),
        jax.ShapeDtypeStruct((NSP, E), jnp.float32),
        jax.ShapeDtypeStruct((NSP, E), jnp.float32),
    ],
    scratch_types=[
        pltpu.VMEM((TPW,), jnp.int32),
        pltpu.VMEM((TPW,), jnp.int32),
        pltpu.VMEM((TPW, E), jnp.float32),
        pltpu.VMEM((TPW, E), jnp.float32),
        pltpu.VMEM((NSP,), jnp.int32),
        pltpu.VMEM((NSP,), jnp.int32),
        pltpu.VMEM((NSP, E), jnp.float32),
        pltpu.SemaphoreType.DMA,
        pltpu.SemaphoreType.DMA,
        pltpu.SemaphoreType.DMA,
    ],
)
def _sc_gather(tgt_hbm, ns_hbm, emb_hbm, bias2d_hbm,
               tb_out, tbrows_out, nb_out, nbrows_out,
               idx_v, ridx_v, rows_v, brows_v,
               nidx_v, nridx_v, nrows_v,
               sem_r, sem_b, nsem):
    wid = lax.axis_index("s") * 2 + lax.axis_index("c")
    base = wid * TPW
    pltpu.sync_copy(tgt_hbm.at[pl.ds(base, TPW)], idx_v)
    cp_rows = pltpu.async_copy(emb_hbm.at[idx_v], rows_v, sem_r)
    _bias_row_idx(idx_v, ridx_v, TPW)
    cp_brows = pltpu.async_copy(bias2d_hbm.at[ridx_v], brows_v, sem_b)

    @pl.when(wid == 0)
    def _():
        pltpu.sync_copy(ns_hbm, nidx_v)
        pltpu.async_copy(emb_hbm.at[nidx_v], nrows_v, nsem).wait()
        pltpu.sync_copy(nrows_v, nb_out)

    @pl.when(wid == 1)
    def _():
        pltpu.sync_copy(ns_hbm, nidx_v)
        _bias_row_idx(nidx_v, nridx_v, NSP)
        pltpu.async_copy(bias2d_hbm.at[nridx_v], nrows_v, nsem).wait()
        pltpu.sync_copy(nrows_v, nbrows_out)

    cp_rows.wait()
    cp_brows.wait()
    pltpu.sync_copy(rows_v, tb_out.at[pl.ds(base, TPW)])
    pltpu.sync_copy(brows_v, tbrows_out.at[pl.ds(base, TPW)])


BLK = 512


def _tc_loss_body(inp_ref, tb_ref, tbrows_ref, tgt_ref, nb_ref, nbrows_ref,
                  ns_ref, out_ref):
    step = pl.program_id(0)

    @pl.when(step == 0)
    def _():
        out_ref[0, 0] = 0.0

    inp_b = inp_ref[...]
    tgt_i = tgt_ref[...]
    lane_t = lax.broadcasted_iota(jnp.int32, (BLK, E), 1)
    tbias = jnp.sum(jnp.where(lane_t == (tgt_i & 127), tbrows_ref[...], 0.0),
                    axis=1, keepdims=True)
    tdot = jnp.sum(inp_b * tb_ref[...], axis=1, keepdims=True) + tbias
    tgt_f = tgt_i.astype(jnp.float32)
    x_t = tdot - LOG_V - (jnp.log(tgt_f + 1.0) - LOG_S) - LOG_NR
    t_sum = jnp.sum(jnp.maximum(-x_t, 0.0)
                    + jnp.log(1.0 + jnp.exp(-jnp.abs(x_t))))

    ns_i = ns_ref[...]
    lane_n = lax.broadcasted_iota(jnp.int32, (NSP, E), 1)
    nbias = jnp.sum(jnp.where(lane_n == (ns_i & 127), nbrows_ref[...], 0.0),
                    axis=1, keepdims=True)
    ns_f = ns_i.astype(jnp.float32)
    row_off = nbias - LOG_V - (jnp.log(ns_f + 1.0) - LOG_S) - LOG_NR
    scores = lax.dot_general(nb_ref[...], inp_b, (((1,), (1,)), ((), ())),
                             preferred_element_type=jnp.float32)
    x_n = scores + row_off
    sp = jnp.maximum(x_n, 0.0) + jnp.log(1.0 + jnp.exp(-jnp.abs(x_n)))
    mask = lax.broadcasted_iota(jnp.int32, (NSP, BLK), 0) < NR
    n_sum = jnp.sum(jnp.where(mask, sp, 0.0))

    out_ref[0, 0] += (t_sum + n_sum) * (1.0 / N)


_tc_loss = pl.pallas_call(
    _tc_loss_body,
    grid=(N // BLK,),
    in_specs=[
        pl.BlockSpec((BLK, E), lambda i: (i, 0)),
        pl.BlockSpec((BLK, E), lambda i: (i, 0)),
        pl.BlockSpec((BLK, E), lambda i: (i, 0)),
        pl.BlockSpec((BLK, 1), lambda i: (i, 0)),
        pl.BlockSpec((NSP, E), lambda i: (0, 0)),
        pl.BlockSpec((NSP, E), lambda i: (0, 0)),
        pl.BlockSpec((NSP, 1), lambda i: (0, 0)),
    ],
    out_specs=pl.BlockSpec(memory_space=pltpu.SMEM),
    out_shape=jax.ShapeDtypeStruct((1, 1), jnp.float32),
)


def kernel(target, inp, noise_samples, emb_weight, bias_weight):
    tgt = target.reshape(-1).astype(jnp.int32)
    ns = jnp.concatenate([noise_samples.astype(jnp.int32),
                          jnp.zeros((NSP - NR,), jnp.int32)])
    inp_flat = inp.reshape(N, E)
    bias2d = jnp.pad(bias_weight.reshape(-1),
                     (0, BROWS * E - V)).reshape(BROWS, E)
    tb, tbrows, nb, nbrows = _sc_gather(tgt, ns, emb_weight, bias2d)
    out = _tc_loss(inp_flat, tb, tbrows, tgt.reshape(N, 1),
                   nb, nbrows, ns.reshape(NSP, 1))
    return out[0, 0]


# speedup vs baseline: 1.8356x; 1.0627x over previous
import functools
import math

import jax
import jax.numpy as jnp
from jax import lax
from jax.experimental import pallas as pl
from jax.experimental.pallas import tpu as pltpu
from jax.experimental.pallas import tpu_sc as plsc

V = 100000
E = 128
B = 128
L = 32
NR = 100
N = B * L
NW = 32
TPW = N // NW
NSP = 112
BROWS = (V + E - 1) // E

LOG_V = math.log(V)
LOG_NR = math.log(NR)
LOG_S = math.log(V * (V + 1) / 2.0)

_mesh = plsc.VectorSubcoreMesh(core_axis_name="c", subcore_axis_name="s")


def _bias_row_idx(idx_ref, out_ref, n):
    for g in range(n // 16):
        out_ref[pl.ds(16 * g, 16)] = lax.shift_right_logical(
            idx_ref[pl.ds(16 * g, 16)], 7)


@functools.partial(
    pl.kernel,
    mesh=_mesh,
    out_type=[
        jax.ShapeDtypeStruct((N, 16), jnp.float32),
        jax.ShapeDtypeStruct((NSP, E), jnp.float32),
        jax.ShapeDtypeStruct((NSP,), jnp.float32),
    ],
    scratch_types=[
        pltpu.VMEM((TPW,), jnp.int32),
        pltpu.VMEM((TPW + 16,), jnp.int32),
        pltpu.VMEM((TPW,), jnp.int32),
        pltpu.VMEM((TPW, E), jnp.float32),
        pltpu.VMEM((TPW, E), jnp.float32),
        pltpu.VMEM((TPW, E), jnp.float32),
        pltpu.VMEM((TPW, 16), jnp.float32),
        pltpu.VMEM((NSP,), jnp.int32),
        pltpu.VMEM((NSP + 16,), jnp.int32),
        pltpu.VMEM((NSP,), jnp.int32),
        pltpu.VMEM((NSP + 1, E), jnp.float32),
        pltpu.VMEM((NSP + 16,), jnp.float32),
        pltpu.SemaphoreType.DMA,
        pltpu.SemaphoreType.DMA,
        pltpu.SemaphoreType.DMA,
        pltpu.SemaphoreType.DMA,
    ],
)
def _sc_gather(tgt_hbm, ns_hbm, inp_hbm, emb_hbm, bias2d_hbm,
               score_out, nb_out, nbias_out,
               idx_v, colv_v, ridx_v, rows_v, brows_v, inp_v, score_v,
               nidx_v, ncolv_v, nridx_v, nrows_v, nbias_v,
               sem_r, sem_b, sem_i, nsem):
    wid = lax.axis_index("s") * 2 + lax.axis_index("c")
    base = wid * TPW
    cp_inp = pltpu.async_copy(inp_hbm.at[pl.ds(base, TPW)], inp_v, sem_i)
    pltpu.sync_copy(tgt_hbm.at[pl.ds(base, TPW)], idx_v)
    cp_rows = pltpu.async_copy(emb_hbm.at[idx_v], rows_v, sem_r)
    _bias_row_idx(idx_v, ridx_v, TPW)
    for g in range(TPW // 16):
        colv_v[pl.ds(16 * g, 16)] = lax.bitwise_and(
            idx_v[pl.ds(16 * g, 16)], 127)
    cp_brows = pltpu.async_copy(bias2d_hbm.at[ridx_v], brows_v, sem_b)

    lanes = lax.iota(jnp.int32, 16)

    @pl.when(wid == 0)
    def _():
        pltpu.sync_copy(ns_hbm, nidx_v)
        pltpu.async_copy(emb_hbm.at[nidx_v],
                         nrows_v.at[pl.ds(0, NSP)], nsem).wait()
        pltpu.sync_copy(nrows_v.at[pl.ds(0, NSP)], nb_out)

    @pl.when(wid == 1)
    def _():
        pltpu.sync_copy(ns_hbm, nidx_v)
        _bias_row_idx(nidx_v, nridx_v, NSP)
        for g in range(NSP // 16):
            ncolv_v[pl.ds(16 * g, 16)] = lax.bitwise_and(
                nidx_v[pl.ds(16 * g, 16)], 127)
        pltpu.async_copy(bias2d_hbm.at[nridx_v],
                         nrows_v.at[pl.ds(0, NSP)], nsem).wait()

        def nbody(i, _):
            col = ncolv_v[pl.ds(i, 16)][0]
            total = nrows_v[i, pl.ds(col, 16)][0]
            nbias_v[pl.ds(i, 16)] = jnp.full((16,), total, jnp.float32)
            return 0

        lax.fori_loop(0, NSP, nbody, 0)
        pltpu.sync_copy(nbias_v.at[pl.ds(0, NSP)], nbias_out)

    cp_inp.wait()
    cp_rows.wait()
    cp_brows.wait()

    def body(i, _):
        acc = rows_v[i, pl.ds(0, 16)] * inp_v[i, pl.ds(0, 16)]
        for j in range(1, E // 16):
            acc = acc + (rows_v[i, pl.ds(16 * j, 16)]
                         * inp_v[i, pl.ds(16 * j, 16)])
        col = colv_v[pl.ds(i, 16)][0]
        bchunk = brows_v[i, pl.ds(col & ~15, 16)]
        acc = acc + jnp.where(lanes == (col & 15), bchunk, 0.0)
        score_v[i, :] = acc
        return 0

    lax.fori_loop(0, TPW, body, 0)
    pltpu.sync_copy(score_v, score_out.at[pl.ds(base, TPW)])


BLK = 512
GR = BLK // 128


def _tc_loss_body(inp_ref, score_ref, tgt_ref, nb_ref, nbias_ref, ns_ref,
                  out_ref):
    step = pl.program_id(0)

    @pl.when(step == 0)
    def _():
        out_ref[0, 0] = 0.0

    tgt_f = tgt_ref[...].astype(jnp.float32)
    tdot = jnp.sum(score_ref[...], axis=3)
    x_t = (tdot - LOG_V
           - (jnp.log(tgt_f + 1.0) - LOG_S) - LOG_NR)
    t_sum = jnp.sum(jnp.maximum(-x_t, 0.0)
                    + jnp.log(1.0 + jnp.exp(-jnp.abs(x_t))))

    ns_f = ns_ref[...].astype(jnp.float32)
    col_off = nbias_ref[...] - LOG_V - (jnp.log(ns_f + 1.0) - LOG_S) - LOG_NR
    scores = lax.dot_general(inp_ref[...], nb_ref[...],
                             (((1,), (1,)), ((), ())),
                             preferred_element_type=jnp.float32)
    x_n = scores + col_off
    sp = jnp.maximum(x_n, 0.0) + jnp.log(1.0 + jnp.exp(-jnp.abs(x_n)))
    mask = lax.broadcasted_iota(jnp.int32, (BLK, NSP), 1) < NR
    n_sum = jnp.sum(jnp.where(mask, sp, 0.0))

    out_ref[0, 0] += (t_sum + n_sum) * (1.0 / N)


_tc_loss = pl.pallas_call(
    _tc_loss_body,
    grid=(N // BLK,),
    in_specs=[
        pl.BlockSpec((BLK, E), lambda i: (i, 0)),
        pl.BlockSpec((1, GR, 128, 16), lambda i: (i, 0, 0, 0)),
        pl.BlockSpec((1, GR, 128), lambda i: (i, 0, 0)),
        pl.BlockSpec((NSP, E), lambda i: (0, 0)),
        pl.BlockSpec((1, NSP), lambda i: (0, 0)),
        pl.BlockSpec((1, NSP), lambda i: (0, 0)),
    ],
    out_specs=pl.BlockSpec(memory_space=pltpu.SMEM),
    out_shape=jax.ShapeDtypeStruct((1, 1), jnp.float32),
)


def kernel(target, inp, noise_samples, emb_weight, bias_weight):
    tgt = target.reshape(-1).astype(jnp.int32)
    ns = jnp.concatenate([noise_samples.astype(jnp.int32),
                          jnp.zeros((NSP - NR,), jnp.int32)])
    inp_flat = inp.reshape(N, E)
    bias2d = jnp.pad(bias_weight.reshape(-1),
                     (0, BROWS * E - V)).reshape(BROWS, E)
    score, nb, nbias = _sc_gather(tgt, ns, inp_flat, emb_weight, bias2d)
    out = _tc_loss(inp_flat, score.reshape(N // BLK, GR, 128, 16),
                   tgt.reshape(N // BLK, GR, 128), nb,
                   nbias.reshape(1, NSP), ns.reshape(1, NSP))
    return out[0, 0]
